# Initial kernel scaffold; baseline (speedup 1.0000x reference)
import jax
import jax.numpy as jnp
from jax.experimental import pallas as pl


def kernel(all_rois, gt_boxes, num_boxes):
    raise NotImplementedError("write your pallas kernel here")


# speedup vs baseline: 9.3391x; 9.3391x over previous
import functools

import jax
import jax.numpy as jnp
from jax import lax
from jax.experimental import pallas as pl
from jax.experimental.pallas import tpu as pltpu

_NJ = 256
_FG_MAX = 64
_FG_THRESH = 0.5
_BG_KEY_OFFSET = 32768.0


def _body(G, M, R, gt_s, x1r, y1r, x2r, y2r, gtm, out_ref, key_sc, aidx_sc):
    f32 = jnp.float32
    x1v = x1r[0]
    y1v = y1r[0]
    x2v = x2r[0]
    y2v = y2r[0]
    area1 = (x2v - x1v + 1.0) * (y2v - y1v + 1.0)

    def g_step(g, carry):
        best, bidx = carry
        gx1 = gt_s[0, 0, g]
        gy1 = gt_s[0, 0, 64 + g]
        gx2 = gt_s[0, 0, 128 + g]
        gy2 = gt_s[0, 0, 192 + g]
        iw = jnp.maximum(jnp.minimum(x2v, gx2) - jnp.maximum(x1v, gx1) + 1.0, 0.0)
        ih = jnp.maximum(jnp.minimum(y2v, gy2) - jnp.maximum(y1v, gy1) + 1.0, 0.0)
        inter = iw * ih
        area2 = (gx2 - gx1 + 1.0) * (gy2 - gy1 + 1.0)
        iou = inter / (area1 + area2 - inter)
        upd = iou > best
        best = jnp.where(upd, iou, best)
        bidx = jnp.where(upd, g.astype(f32), bidx)
        return best, bidx

    mo, aidxf = lax.fori_loop(
        0, G, g_step,
        (jnp.full((R, 128), -1.0, f32), jnp.zeros((R, 128), f32)))

    ridx = lax.broadcasted_iota(jnp.int32, (R, 128), 0)
    cidx = lax.broadcasted_iota(jnp.int32, (R, 128), 1)
    valid = (ridx * 128 + cidx) < M
    fg = (mo >= _FG_THRESH) & valid
    bg = (mo < _FG_THRESH) & (mo >= 0.0) & valid
    fgf = fg.astype(f32)
    bgf = bg.astype(f32)

    li = lax.broadcasted_iota(jnp.int32, (128, 128), 0)
    lj = lax.broadcasted_iota(jnp.int32, (128, 128), 1)
    tri_inc = (li <= lj).astype(f32)
    ri = lax.broadcasted_iota(jnp.int32, (R, R), 0)
    rj = lax.broadcasted_iota(jnp.int32, (R, R), 1)
    tri_exc = (rj < ri).astype(f32)

    def ranks(maskf):
        csr = lax.dot_general(maskf, tri_inc, (((1,), (0,)), ((), ())),
                              preferred_element_type=f32)
        rowsum = csr[:, 127:128]
        offs = lax.dot_general(tri_exc, rowsum, (((1,), (0,)), ((), ())),
                               preferred_element_type=f32)
        return csr + offs - 1.0

    rank_fg = ranks(fgf)
    rank_bg = ranks(bgf)
    fgn_i = jnp.sum(fgf).astype(jnp.int32)
    bgn_i = jnp.sum(bgf).astype(jnp.int32)

    key = jnp.where(fg, rank_fg, jnp.where(bg, rank_bg + _BG_KEY_OFFSET, -5.0))
    key_sc[...] = key
    aidx_sc[...] = aidxf

    fg_this = jnp.where(
        fgn_i > 0,
        jnp.where(bgn_i > 0, jnp.minimum(jnp.int32(_FG_MAX), fgn_i),
                  jnp.int32(_NJ)),
        jnp.int32(0))
    jj = lax.broadcasted_iota(jnp.int32, (_NJ, 1), 0)
    tfg = lax.rem(jj, jnp.maximum(fgn_i, 1))
    tbg = lax.rem(jj - fg_this, jnp.maximum(bgn_i, 1))
    is_fg_slot = jj < fg_this
    tt = jnp.where(is_fg_slot, tfg,
                   tbg + jnp.int32(int(_BG_KEY_OFFSET))).astype(f32)

    def r_step(r, acc):
        keyrow = key_sc[pl.ds(r, 1), :]
        sel = (tt == keyrow).astype(f32)
        fx1 = x1r[0, pl.ds(r, 1), :]
        fy1 = y1r[0, pl.ds(r, 1), :]
        fx2 = x2r[0, pl.ds(r, 1), :]
        fy2 = y2r[0, pl.ds(r, 1), :]
        fa = aidx_sc[pl.ds(r, 1), :]
        feats = jnp.concatenate(
            [fx1, fy1, fx2, fy2, fa, jnp.zeros((3, 128), f32)], axis=0)
        return acc + lax.dot_general(sel, feats, (((1,), (1,)), ((), ())),
                                     precision=lax.Precision.HIGHEST,
                                     preferred_element_type=f32)

    acc = lax.fori_loop(0, R, r_step, jnp.zeros((_NJ, 8), f32))

    sx1 = acc[:, 0:1]
    sy1 = acc[:, 1:2]
    sx2 = acc[:, 2:3]
    sy2 = acc[:, 3:4]
    sa = acc[:, 4:5]

    giota = lax.broadcasted_iota(jnp.int32, (1, 64), 1).astype(f32)
    onehot = (sa == giota).astype(f32)
    gtr = lax.dot_general(onehot, gtm[0], (((1,), (1,)), ((), ())),
                          precision=lax.Precision.HIGHEST,
                          preferred_element_type=f32)
    gx1 = gtr[:, 0:1]
    gy1 = gtr[:, 1:2]
    gx2 = gtr[:, 2:3]
    gy2 = gtr[:, 3:4]
    gcls = gtr[:, 4:5]

    ex_w = sx2 - sx1 + 1.0
    ex_h = sy2 - sy1 + 1.0
    ex_cx = sx1 + 0.5 * ex_w
    ex_cy = sy1 + 0.5 * ex_h
    gt_w = gx2 - gx1 + 1.0
    gt_h = gy2 - gy1 + 1.0
    gt_cx = gx1 + 0.5 * gt_w
    gt_cy = gy1 + 0.5 * gt_h
    dx = (gt_cx - ex_cx) / ex_w
    dy = (gt_cy - ex_cy) / ex_h
    dw = jnp.log(gt_w / ex_w)
    dh = jnp.log(gt_h / ex_h)

    labelz = jnp.where(is_fg_slot, gcls, 0.0)
    fgm = labelz > 0.0
    tx = jnp.where(fgm, dx / 0.1, 0.0)
    ty = jnp.where(fgm, dy / 0.1, 0.0)
    tw = jnp.where(fgm, dw / 0.2, 0.0)
    th = jnp.where(fgm, dh / 0.2, 0.0)
    inw = jnp.where(fgm, 1.0, 0.0)

    outm = jnp.concatenate(
        [sx1, sy1, sx2, sy2, labelz, tx, ty, tw, th,
         inw, inw, inw, inw, inw, inw, inw, inw,
         jnp.zeros((_NJ, 7), f32)], axis=1)
    out_ref[0] = outm


def kernel(all_rois, gt_boxes, num_boxes):
    B, N, _ = all_rois.shape
    G = gt_boxes.shape[1]
    M = N + G
    R = -(-M // 128)
    Mpad = R * 128

    coords = jnp.concatenate([all_rois[:, :, 1:5], gt_boxes[:, :, :4]], axis=1)
    coords = jnp.pad(coords, ((0, 0), (0, Mpad - M), (0, 0)))
    ct = coords.transpose(0, 2, 1).reshape(B, 4, R, 128)
    x1, y1, x2, y2 = ct[:, 0], ct[:, 1], ct[:, 2], ct[:, 3]

    gt_t = jnp.swapaxes(gt_boxes, 1, 2)
    gt_pad = jnp.pad(gt_t, ((0, 0), (0, 0), (0, 64 - G)))
    gt_sm = gt_pad.reshape(B, 1, 320)
    gt_mm = jnp.pad(gt_pad, ((0, 0), (0, 3), (0, 0)))

    body = functools.partial(_body, G, M, R)
    out = pl.pallas_call(
        body,
        grid=(B,),
        in_specs=[
            pl.BlockSpec((1, 1, 320), lambda b: (b, 0, 0),
                         memory_space=pltpu.SMEM),
            pl.BlockSpec((1, R, 128), lambda b: (b, 0, 0)),
            pl.BlockSpec((1, R, 128), lambda b: (b, 0, 0)),
            pl.BlockSpec((1, R, 128), lambda b: (b, 0, 0)),
            pl.BlockSpec((1, R, 128), lambda b: (b, 0, 0)),
            pl.BlockSpec((1, 8, 64), lambda b: (b, 0, 0)),
        ],
        out_specs=pl.BlockSpec((1, _NJ, 24), lambda b: (b, 0, 0)),
        out_shape=jax.ShapeDtypeStruct((B, _NJ, 24), jnp.float32),
        scratch_shapes=[
            pltpu.VMEM((R, 128), jnp.float32),
            pltpu.VMEM((R, 128), jnp.float32),
        ],
    )(gt_sm, x1, y1, x2, y2, gt_mm)

    bcol = jnp.broadcast_to(
        jnp.arange(B, dtype=jnp.float32)[:, None, None], (B, _NJ, 1))
    rois = jnp.concatenate([bcol, out[:, :, 0:4]], axis=2)
    labels = out[:, :, 4]
    targets = out[:, :, 5:9]
    inside_w = out[:, :, 9:13]
    outside_w = out[:, :, 13:17]
    return rois, labels, targets, inside_w, outside_w


# speedup vs baseline: 28.3340x; 3.0339x over previous
import functools

import jax
import jax.numpy as jnp
from jax import lax
from jax.experimental import pallas as pl
from jax.experimental.pallas import tpu as pltpu

_NJ = 256
_FG_MAX = 64
_FG_THRESH = 0.5
_BG_KEY_OFFSET = 32768.0


def _body(G, M, R, gt_s, x1r, y1r, x2r, y2r, gtm, out_ref, key_sc, aidx_sc):
    f32 = jnp.float32
    x1v = x1r[0]
    y1v = y1r[0]
    x2v = x2r[0]
    y2v = y2r[0]
    area1 = (x2v - x1v + 1.0) * (y2v - y1v + 1.0)

    def g_step(g, carry):
        best, bidx = carry
        gx1 = gt_s[0, 0, g]
        gy1 = gt_s[0, 0, 64 + g]
        gx2 = gt_s[0, 0, 128 + g]
        gy2 = gt_s[0, 0, 192 + g]
        iw = jnp.maximum(jnp.minimum(x2v, gx2) - jnp.maximum(x1v, gx1) + 1.0, 0.0)
        ih = jnp.maximum(jnp.minimum(y2v, gy2) - jnp.maximum(y1v, gy1) + 1.0, 0.0)
        inter = iw * ih
        area2 = (gx2 - gx1 + 1.0) * (gy2 - gy1 + 1.0)
        iou = inter / (area1 + area2 - inter)
        upd = iou > best
        best = jnp.where(upd, iou, best)
        bidx = jnp.where(upd, g.astype(f32), bidx)
        return best, bidx

    mo, aidxf = lax.fori_loop(
        0, G, g_step,
        (jnp.full((R, 128), -1.0, f32), jnp.zeros((R, 128), f32)))

    ridx = lax.broadcasted_iota(jnp.int32, (R, 128), 0)
    cidx = lax.broadcasted_iota(jnp.int32, (R, 128), 1)
    valid = (ridx * 128 + cidx) < M
    fg = (mo >= _FG_THRESH) & valid
    bg = (mo < _FG_THRESH) & (mo >= 0.0) & valid
    fgf = fg.astype(f32)
    bgf = bg.astype(f32)

    li = lax.broadcasted_iota(jnp.int32, (128, 128), 0)
    lj = lax.broadcasted_iota(jnp.int32, (128, 128), 1)
    tri_inc = (li <= lj).astype(f32)
    ri = lax.broadcasted_iota(jnp.int32, (R, R), 0)
    rj = lax.broadcasted_iota(jnp.int32, (R, R), 1)
    tri_exc = (rj < ri).astype(f32)

    def ranks(maskf):
        csr = lax.dot_general(maskf, tri_inc, (((1,), (0,)), ((), ())),
                              preferred_element_type=f32)
        rowsum = csr[:, 127:128]
        offs = lax.dot_general(tri_exc, rowsum, (((1,), (0,)), ((), ())),
                               preferred_element_type=f32)
        return csr + offs - 1.0

    rank_fg = ranks(fgf)
    rank_bg = ranks(bgf)
    fgn_i = jnp.sum(fgf).astype(jnp.int32)
    bgn_i = jnp.sum(bgf).astype(jnp.int32)

    key = jnp.where(fg, rank_fg, jnp.where(bg, rank_bg + _BG_KEY_OFFSET, -5.0))
    key_sc[...] = key
    aidx_sc[...] = aidxf

    fg_this = jnp.where(
        fgn_i > 0,
        jnp.where(bgn_i > 0, jnp.minimum(jnp.int32(_FG_MAX), fgn_i),
                  jnp.int32(_NJ)),
        jnp.int32(0))
    jj = lax.broadcasted_iota(jnp.int32, (_NJ, 1), 0)
    tfg = lax.rem(jj, jnp.maximum(fgn_i, 1))
    tbg = lax.rem(jj - fg_this, jnp.maximum(bgn_i, 1))
    is_fg_slot = jj < fg_this
    tt = jnp.where(is_fg_slot, tfg,
                   tbg + jnp.int32(int(_BG_KEY_OFFSET))).astype(f32)

    laneiota = lax.broadcasted_iota(jnp.int32, (1, 128), 1).astype(f32)

    def r_step(r, acc):
        keyrow = key_sc[pl.ds(r, 1), :]
        sel = tt == keyrow
        mrow = laneiota + r.astype(f32) * 128.0
        return jnp.where(sel, mrow, acc)

    acc = lax.fori_loop(0, R, r_step, jnp.zeros((_NJ, 128), f32))
    keep = jnp.sum(acc, axis=1, keepdims=True)
    rj = jnp.floor(keep * (1.0 / 128.0))
    cj = keep - rj * 128.0

    riota = lax.broadcasted_iota(jnp.int32, (1, R), 1).astype(f32)
    rowsel = (rj == riota).astype(f32)
    featsall = jnp.concatenate([x1v, y1v, x2v, y2v, aidx_sc[...]], axis=1)
    gathered = lax.dot_general(rowsel, featsall, (((1,), (0,)), ((), ())),
                               precision=lax.Precision.HIGHEST,
                               preferred_element_type=f32)
    lanesel = cj == laneiota

    def lanepick(f):
        seg = gathered[:, f * 128:(f + 1) * 128]
        return jnp.sum(jnp.where(lanesel, seg, 0.0), axis=1, keepdims=True)

    sx1 = lanepick(0)
    sy1 = lanepick(1)
    sx2 = lanepick(2)
    sy2 = lanepick(3)
    sa = lanepick(4)

    giota = lax.broadcasted_iota(jnp.int32, (1, 64), 1).astype(f32)
    onehot = (sa == giota).astype(f32)
    gtr = lax.dot_general(onehot, gtm[0], (((1,), (1,)), ((), ())),
                          precision=lax.Precision.HIGHEST,
                          preferred_element_type=f32)
    gx1 = gtr[:, 0:1]
    gy1 = gtr[:, 1:2]
    gx2 = gtr[:, 2:3]
    gy2 = gtr[:, 3:4]
    gcls = gtr[:, 4:5]

    ex_w = sx2 - sx1 + 1.0
    ex_h = sy2 - sy1 + 1.0
    ex_cx = sx1 + 0.5 * ex_w
    ex_cy = sy1 + 0.5 * ex_h
    gt_w = gx2 - gx1 + 1.0
    gt_h = gy2 - gy1 + 1.0
    gt_cx = gx1 + 0.5 * gt_w
    gt_cy = gy1 + 0.5 * gt_h
    dx = (gt_cx - ex_cx) / ex_w
    dy = (gt_cy - ex_cy) / ex_h
    dw = jnp.log(gt_w / ex_w)
    dh = jnp.log(gt_h / ex_h)

    labelz = jnp.where(is_fg_slot, gcls, 0.0)
    fgm = labelz > 0.0
    tx = jnp.where(fgm, dx / 0.1, 0.0)
    ty = jnp.where(fgm, dy / 0.1, 0.0)
    tw = jnp.where(fgm, dw / 0.2, 0.0)
    th = jnp.where(fgm, dh / 0.2, 0.0)
    inw = jnp.where(fgm, 1.0, 0.0)

    outm = jnp.concatenate(
        [sx1, sy1, sx2, sy2, labelz, tx, ty, tw, th,
         inw, inw, inw, inw, inw, inw, inw, inw,
         jnp.zeros((_NJ, 7), f32)], axis=1)
    out_ref[0] = outm


def kernel(all_rois, gt_boxes, num_boxes):
    B, N, _ = all_rois.shape
    G = gt_boxes.shape[1]
    M = N + G
    R = -(-M // 128)
    Mpad = R * 128

    coords = jnp.concatenate([all_rois[:, :, 1:5], gt_boxes[:, :, :4]], axis=1)
    coords = jnp.pad(coords, ((0, 0), (0, Mpad - M), (0, 0)))
    ct = coords.transpose(0, 2, 1).reshape(B, 4, R, 128)
    x1, y1, x2, y2 = ct[:, 0], ct[:, 1], ct[:, 2], ct[:, 3]

    gt_t = jnp.swapaxes(gt_boxes, 1, 2)
    gt_pad = jnp.pad(gt_t, ((0, 0), (0, 0), (0, 64 - G)))
    gt_sm = gt_pad.reshape(B, 1, 320)
    gt_mm = jnp.pad(gt_pad, ((0, 0), (0, 3), (0, 0)))

    body = functools.partial(_body, G, M, R)
    out = pl.pallas_call(
        body,
        grid=(B,),
        in_specs=[
            pl.BlockSpec((1, 1, 320), lambda b: (b, 0, 0),
                         memory_space=pltpu.SMEM),
            pl.BlockSpec((1, R, 128), lambda b: (b, 0, 0)),
            pl.BlockSpec((1, R, 128), lambda b: (b, 0, 0)),
            pl.BlockSpec((1, R, 128), lambda b: (b, 0, 0)),
            pl.BlockSpec((1, R, 128), lambda b: (b, 0, 0)),
            pl.BlockSpec((1, 8, 64), lambda b: (b, 0, 0)),
        ],
        out_specs=pl.BlockSpec((1, _NJ, 24), lambda b: (b, 0, 0)),
        out_shape=jax.ShapeDtypeStruct((B, _NJ, 24), jnp.float32),
        scratch_shapes=[
            pltpu.VMEM((R, 128), jnp.float32),
            pltpu.VMEM((R, 128), jnp.float32),
        ],
    )(gt_sm, x1, y1, x2, y2, gt_mm)

    bcol = jnp.broadcast_to(
        jnp.arange(B, dtype=jnp.float32)[:, None, None], (B, _NJ, 1))
    rois = jnp.concatenate([bcol, out[:, :, 0:4]], axis=2)
    labels = out[:, :, 4]
    targets = out[:, :, 5:9]
    inside_w = out[:, :, 9:13]
    outside_w = out[:, :, 13:17]
    return rois, labels, targets, inside_w, outside_w
